# baseline (speedup 1.0000x reference)
import functools

import jax
import jax.numpy as jnp
from jax import lax
from jax.experimental import pallas as pl
from jax.experimental.pallas import tpu as pltpu
from jax.experimental.pallas import tpu_sc as plsc

N = 10000
E = 160000
D = 256
QF = 64
QW = 80
NC = 2
NS = 16
LANES = 16
B = 128
NB_PT = 81
E_PAD = NB_PT * NS * B
ROWS_PAD = E_PAD // B
NBUF = 3
NPT = N // NS
BLK = 1000
NBLK = N // BLK

_sc_mesh = plsc.VectorSubcoreMesh(
    core_axis_name="c", subcore_axis_name="s", num_cores=NC, num_subcores=NS)


@functools.partial(
    pl.kernel,
    out_type=jax.ShapeDtypeStruct((4 * N, QW), jnp.float32),
    mesh=_sc_mesh,
    scratch_types=[
        pltpu.VMEM_SHARED((N, QW), jnp.float32),
        pltpu.VMEM((NB_PT, 3, B), jnp.int32),
        pltpu.VMEM((NB_PT, B), jnp.float32),
    ] + [pltpu.VMEM((B, QW), jnp.float32) for _ in range(NBUF)]
      + [pltpu.SemaphoreType.DMA for _ in range(2 * NBUF)],
    compiler_params=pltpu.CompilerParams(use_tc_tiling_on_sc=False),
)
def _sc_agg(table_a, table_b, packed_hbm, w_hbm, zeros_hbm, out_hbm,
            acc, pidx, wts, *scr):
  rows = scr[0:NBUF]
  gsem = scr[NBUF:2 * NBUF]
  ssem = scr[2 * NBUF:3 * NBUF]
  c = lax.axis_index("c")
  s = lax.axis_index("s")

  pltpu.sync_copy(packed_hbm.at[pl.ds(s * NB_PT, NB_PT)], pidx)
  pltpu.sync_copy(w_hbm.at[pl.ds(s * NB_PT, NB_PT)], wts)

  def scale(j, b):
    rj = rows[j]

    def mul_chunk(kb, carry):
      kbase = kb * LANES
      wk_vec = wts[b, pl.ds(kbase, LANES)]
      for l in range(LANES):
        wk = wk_vec[l]
        for q in range(QW // LANES):
          sl = pl.ds(q * LANES, LANES)
          rj[kbase + l, sl] = rj[kbase + l, sl] * wk
      return carry

    lax.fori_loop(0, B // LANES, mul_chunk, 0)

  def scatter_start(j, b):
    pltpu.async_copy(rows[j], acc.at[pidx.at[b, 2]], ssem[j], add=True)

  def scatter_wait(j, b):
    pltpu.make_async_copy(rows[j], acc.at[pidx.at[b, 2]], ssem[j]).wait()

  for ph, tab in enumerate((table_a, table_b)):
    def gather_start(j, b, tab=tab):
      pltpu.async_copy(tab.at[pidx.at[b, c]], rows[j], gsem[j])

    def gather_wait(j, b, tab=tab):
      pltpu.make_async_copy(tab.at[pidx.at[b, c]], rows[j], gsem[j]).wait()

    pltpu.sync_copy(zeros_hbm, acc.at[pl.ds(s * NPT, NPT)])
    plsc.subcore_barrier()

    gather_start(0, 0)
    gather_start(1, 1)

    def body(i, carry, gather_start=gather_start, gather_wait=gather_wait):
      for j in range(NBUF):
        bl = NBUF * i + j
        gather_wait(j, bl)
        scale(j, bl)
        scatter_start(j, bl)
        p = (j + 2) % NBUF
        bn = bl + 2

        @pl.when(bn < NB_PT)
        def _():
          @pl.when(bn >= NBUF)
          def _():
            scatter_wait(p, bn - NBUF)
          gather_start(p, bn)

      return carry

    lax.fori_loop(0, NB_PT // NBUF, body, 0)
    for j in range(NBUF):
      scatter_wait(j, NB_PT - NBUF + j)
    plsc.subcore_barrier()
    pltpu.sync_copy(acc.at[pl.ds(s * NPT, NPT)],
                    out_hbm.at[pl.ds((c * 2 + ph) * N + s * NPT, NPT)])
    plsc.subcore_barrier()


def _tc_layer1(a1, w1, b1):
  def body(q0_ref, q1_ref, q2_ref, q3_ref, w1_ref, b1_ref, tc_ref, td_ref):
    q0 = q0_ref[...]
    deg = q0[:, QF:QF + 1]
    sc = jnp.where(deg > 0.0, 1.0 / deg, 0.0)
    x = jnp.concatenate(
        [q0[:, :QF], q1_ref[:, :QF], q2_ref[:, :QF], q3_ref[:, :QF]],
        axis=1) * sc
    h = (jnp.dot(x, w1_ref[...], preferred_element_type=jnp.float32)
         + b1_ref[...])
    h = jnp.maximum(h, 0.0)
    z = jnp.zeros((BLK, QW - QF), jnp.float32)
    tc_ref[...] = jnp.concatenate([h[:, :QF], z], axis=1)
    td_ref[...] = jnp.concatenate([h[:, QF:], z], axis=1)

  return pl.pallas_call(
      body,
      grid=(2, NBLK),
      in_specs=[
          pl.BlockSpec((BLK, QW), lambda jh, i: (0 * NBLK + i, 0)),
          pl.BlockSpec((BLK, QW), lambda jh, i: (1 * NBLK + i, 0)),
          pl.BlockSpec((BLK, QW), lambda jh, i: (2 * NBLK + i, 0)),
          pl.BlockSpec((BLK, QW), lambda jh, i: (3 * NBLK + i, 0)),
          pl.BlockSpec((D, 2 * QF), lambda jh, i: (0, jh)),
          pl.BlockSpec((1, 2 * QF), lambda jh, i: (0, jh)),
      ],
      out_specs=[
          pl.BlockSpec((BLK, QW), lambda jh, i: (jh * NBLK + i, 0)),
          pl.BlockSpec((BLK, QW), lambda jh, i: (jh * NBLK + i, 0)),
      ],
      out_shape=[
          jax.ShapeDtypeStruct((2 * N, QW), jnp.float32),
          jax.ShapeDtypeStruct((2 * N, QW), jnp.float32),
      ],
      compiler_params=pltpu.CompilerParams(
          dimension_semantics=("parallel", "parallel")),
  )(a1, a1, a1, a1, w1, b1.reshape(1, D))


def _tc_layer2(a2, a1, w2, b2, wd, bd, wc, bc):
  def body(q0_ref, q1_ref, q2_ref, q3_ref, dega_ref, w2_ref, b2_ref,
           wd_ref, bd_ref, wc_ref, bc_ref, out_ref, acc_ref):
    i = pl.program_id(0)

    @pl.when(i == 0)
    def _():
      acc_ref[...] = jnp.zeros_like(acc_ref)

    deg = dega_ref[:, QF:QF + 1]
    sc = jnp.where(deg > 0.0, 1.0 / deg, 0.0)
    x = jnp.concatenate(
        [q0_ref[:, :QF], q1_ref[:, :QF], q2_ref[:, :QF], q3_ref[:, :QF]],
        axis=1) * sc
    h2 = (jnp.dot(x, w2_ref[...], preferred_element_type=jnp.float32)
          + b2_ref[...])
    h2 = jnp.maximum(h2, 0.0)
    acc_ref[...] += jnp.sum(h2, axis=0, keepdims=True)

    @pl.when(i == NBLK - 1)
    def _():
      hg = acc_ref[...] * (1.0 / N)
      o1 = jnp.maximum(
          jnp.dot(hg, wd_ref[...], preferred_element_type=jnp.float32)
          + bd_ref[...], 0.0)
      out_ref[...] = (
          jnp.dot(o1, wc_ref[...], preferred_element_type=jnp.float32)
          + bc_ref[...])

  return pl.pallas_call(
      body,
      grid=(NBLK,),
      in_specs=[
          pl.BlockSpec((BLK, QW), lambda i: (0 * NBLK + i, 0)),
          pl.BlockSpec((BLK, QW), lambda i: (1 * NBLK + i, 0)),
          pl.BlockSpec((BLK, QW), lambda i: (2 * NBLK + i, 0)),
          pl.BlockSpec((BLK, QW), lambda i: (3 * NBLK + i, 0)),
          pl.BlockSpec((BLK, QW), lambda i: (i, 0)),
          pl.BlockSpec((D, D), lambda i: (0, 0)),
          pl.BlockSpec((1, D), lambda i: (0, 0)),
          pl.BlockSpec((D, 2 * QF), lambda i: (0, 0)),
          pl.BlockSpec((1, 2 * QF), lambda i: (0, 0)),
          pl.BlockSpec((2 * QF, 10), lambda i: (0, 0)),
          pl.BlockSpec((1, 10), lambda i: (0, 0)),
      ],
      out_specs=pl.BlockSpec((1, 10), lambda i: (0, 0)),
      out_shape=jax.ShapeDtypeStruct((1, 10), jnp.float32),
      scratch_shapes=[pltpu.VMEM((1, D), jnp.float32)],
      compiler_params=pltpu.CompilerParams(
          dimension_semantics=("arbitrary",)),
  )(a2, a2, a2, a2, a1, w2, b2.reshape(1, D), wd, bd.reshape(1, 2 * QF),
    wc, bc.reshape(1, 10))


def kernel(in_feat, edge_weights, W1, b1, W2, b2, Wd, bd, Wc, bc, edge_index):
  npad = E_PAD - E
  src = jnp.concatenate([edge_index[0], jnp.zeros((npad,), jnp.int32)])
  dst = jnp.concatenate([edge_index[1], jnp.zeros((npad,), jnp.int32)])
  w = jnp.concatenate([edge_weights, jnp.zeros((npad,), jnp.float32)])
  packed = jnp.stack([src, src + N, dst], axis=0)
  packed = packed.reshape(3, ROWS_PAD, B).transpose(1, 0, 2)
  w_rows = w.reshape(ROWS_PAD, B)

  ones = jnp.ones((N, 1), jnp.float32)
  pad = jnp.zeros((N, QW - QF - 1), jnp.float32)
  tab_a = jnp.concatenate([
      jnp.concatenate([in_feat[:, 0 * QF:1 * QF], ones, pad], axis=1),
      jnp.concatenate([in_feat[:, 2 * QF:3 * QF], ones, pad], axis=1),
  ], axis=0)
  tab_b = jnp.concatenate([
      jnp.concatenate([in_feat[:, 1 * QF:2 * QF], ones, pad], axis=1),
      jnp.concatenate([in_feat[:, 3 * QF:4 * QF], ones, pad], axis=1),
  ], axis=0)

  zeros_q = jnp.zeros((NPT, QW), jnp.float32)

  a1 = _sc_agg(tab_a, tab_b, packed, w_rows, zeros_q)
  tab_c, tab_d = _tc_layer1(a1, W1, b1)
  a2 = _sc_agg(tab_c, tab_d, packed, w_rows, zeros_q)
  return _tc_layer2(a2, a1, W2, b2, Wd, bd, Wc, bc)


# speedup vs baseline: 1.3266x; 1.3266x over previous
import functools

import jax
import jax.numpy as jnp
from jax import lax
from jax.experimental import pallas as pl
from jax.experimental.pallas import tpu as pltpu
from jax.experimental.pallas import tpu_sc as plsc

N = 10000
E = 160000
D = 256
HALF = 128
AUGW = 144
NC = 2
NS = 16
LANES = 16
B = 128
NB_PT = 80
E_PAD = NB_PT * NS * B
ROWS_PAD = E_PAD // B
NPT = N // NS
BLK = 1000
NBLK = N // BLK


def _make_sc_aggregate(width):
  mesh = plsc.VectorSubcoreMesh(
      core_axis_name="c", subcore_axis_name="s", num_cores=NC, num_subcores=NS)

  @functools.partial(
      pl.kernel,
      out_type=jax.ShapeDtypeStruct((NC * N, width), jnp.float32),
      mesh=mesh,
      scratch_types=[
          pltpu.VMEM_SHARED((N, width), jnp.float32),
      ] + [pltpu.VMEM((B, width), jnp.float32) for _ in range(2)]
        + [pltpu.VMEM((3, B), jnp.int32) for _ in range(2)]
        + [pltpu.VMEM((B,), jnp.float32) for _ in range(2)]
        + [pltpu.SemaphoreType.DMA for _ in range(6)],
      compiler_params=pltpu.CompilerParams(use_tc_tiling_on_sc=False),
  )
  def agg(table_hbm, packed_hbm, w_hbm, zeros_hbm, out_hbm, acc, *scr):
    rows = scr[0:2]
    idxb = scr[2:4]
    wb = scr[4:6]
    gsem = scr[6:8]
    ssem = scr[8:10]
    isem = scr[10:12]
    c = lax.axis_index("c")
    s = lax.axis_index("s")
    base_r = s * NB_PT

    pltpu.sync_copy(zeros_hbm, acc.at[pl.ds(s * NPT, NPT)])
    plsc.subcore_barrier()

    def idx_start(j, b):
      r = base_r + b
      pltpu.async_copy(packed_hbm.at[r], idxb[j], isem[j])
      pltpu.async_copy(w_hbm.at[r], wb[j], isem[j])

    def idx_wait(j, b):
      r = base_r + b
      pltpu.make_async_copy(packed_hbm.at[r], idxb[j], isem[j]).wait()
      pltpu.make_async_copy(w_hbm.at[r], wb[j], isem[j]).wait()

    def gather_start(j):
      pltpu.async_copy(table_hbm.at[idxb[j].at[c]], rows[j], gsem[j])

    def gather_wait(j):
      pltpu.make_async_copy(table_hbm.at[idxb[j].at[c]], rows[j],
                            gsem[j]).wait()

    def scatter_start(j):
      pltpu.async_copy(rows[j], acc.at[idxb[j].at[2]], ssem[j], add=True)

    def scatter_wait(j):
      pltpu.make_async_copy(rows[j], acc.at[idxb[j].at[2]], ssem[j]).wait()

    def scale(j):
      rj = rows[j]
      wj = wb[j]

      def mul_chunk(kb, carry):
        kbase = kb * LANES
        wk_vec = wj[pl.ds(kbase, LANES)]
        for l in range(LANES):
          wk = wk_vec[l]
          for q in range(width // LANES):
            sl = pl.ds(q * LANES, LANES)
            rj[kbase + l, sl] = rj[kbase + l, sl] * wk
        return carry

      lax.fori_loop(0, B // LANES, mul_chunk, 0)

    idx_start(0, 0)
    idx_wait(0, 0)
    gather_start(0)

    def body(i, carry):
      for j in range(2):
        b = 2 * i + j
        p = 1 - j
        gather_wait(j)

        @pl.when(b - 1 >= 0)
        def _():
          scatter_wait(p)

        @pl.when(b + 1 < NB_PT)
        def _():
          idx_start(p, b + 1)

        scale(j)

        @pl.when(b + 1 < NB_PT)
        def _():
          idx_wait(p, b + 1)
          gather_start(p)

        scatter_start(j)
      return carry

    lax.fori_loop(0, NB_PT // 2, body, 0)
    scatter_wait(1)
    plsc.subcore_barrier()
    pltpu.sync_copy(acc.at[pl.ds(s * NPT, NPT)],
                    out_hbm.at[pl.ds(c * N + s * NPT, NPT)])

  return agg


_sc_agg_aug = _make_sc_aggregate(AUGW)
_sc_agg_half = _make_sc_aggregate(HALF)


def _tc_layer1(a1, w1, b1):
  def body(aa_ref, ab_ref, w1a_ref, w1b_ref, b1_ref, out_ref):
    aa = aa_ref[...]
    ab = ab_ref[...]
    deg = aa[:, HALF:HALF + 1]
    sc = jnp.where(deg > 0.0, 1.0 / deg, 0.0)
    xa = aa[:, :HALF] * sc
    xb = ab[:, :HALF] * sc
    h = (jnp.dot(xa, w1a_ref[...], preferred_element_type=jnp.float32)
         + jnp.dot(xb, w1b_ref[...], preferred_element_type=jnp.float32)
         + b1_ref[...])
    out_ref[...] = jnp.maximum(h, 0.0)

  return pl.pallas_call(
      body,
      grid=(2, NBLK),
      in_specs=[
          pl.BlockSpec((BLK, AUGW), lambda j, i: (i, 0)),
          pl.BlockSpec((BLK, AUGW), lambda j, i: (i + NBLK, 0)),
          pl.BlockSpec((HALF, HALF), lambda j, i: (0, j)),
          pl.BlockSpec((HALF, HALF), lambda j, i: (1, j)),
          pl.BlockSpec((1, HALF), lambda j, i: (0, j)),
      ],
      out_specs=pl.BlockSpec((BLK, HALF), lambda j, i: (j * NBLK + i, 0)),
      out_shape=jax.ShapeDtypeStruct((2 * N, HALF), jnp.float32),
      compiler_params=pltpu.CompilerParams(
          dimension_semantics=("parallel", "parallel")),
  )(a1, a1, w1, w1, b1.reshape(1, D))


def _tc_layer2(a2, a1, w2, b2, wd, bd, wc, bc):
  def body(a2a_ref, a2b_ref, dega_ref, w2a_ref, w2b_ref, b2_ref,
           wd_ref, bd_ref, wc_ref, bc_ref, out_ref, acc_ref):
    i = pl.program_id(0)

    @pl.when(i == 0)
    def _():
      acc_ref[...] = jnp.zeros_like(acc_ref)

    deg = dega_ref[...][:, HALF:HALF + 1]
    sc = jnp.where(deg > 0.0, 1.0 / deg, 0.0)
    xa = a2a_ref[...] * sc
    xb = a2b_ref[...] * sc
    h2 = (jnp.dot(xa, w2a_ref[...], preferred_element_type=jnp.float32)
          + jnp.dot(xb, w2b_ref[...], preferred_element_type=jnp.float32)
          + b2_ref[...])
    h2 = jnp.maximum(h2, 0.0)
    acc_ref[...] += jnp.sum(h2, axis=0, keepdims=True)

    @pl.when(i == NBLK - 1)
    def _():
      hg = acc_ref[...] * (1.0 / N)
      o1 = jnp.maximum(
          jnp.dot(hg, wd_ref[...], preferred_element_type=jnp.float32)
          + bd_ref[...], 0.0)
      out_ref[...] = (
          jnp.dot(o1, wc_ref[...], preferred_element_type=jnp.float32)
          + bc_ref[...])

  return pl.pallas_call(
      body,
      grid=(NBLK,),
      in_specs=[
          pl.BlockSpec((BLK, HALF), lambda i: (i, 0)),
          pl.BlockSpec((BLK, HALF), lambda i: (i + NBLK, 0)),
          pl.BlockSpec((BLK, AUGW), lambda i: (i, 0)),
          pl.BlockSpec((HALF, D), lambda i: (0, 0)),
          pl.BlockSpec((HALF, D), lambda i: (1, 0)),
          pl.BlockSpec((1, D), lambda i: (0, 0)),
          pl.BlockSpec((D, HALF), lambda i: (0, 0)),
          pl.BlockSpec((1, HALF), lambda i: (0, 0)),
          pl.BlockSpec((HALF, 10), lambda i: (0, 0)),
          pl.BlockSpec((1, 10), lambda i: (0, 0)),
      ],
      out_specs=pl.BlockSpec((1, 10), lambda i: (0, 0)),
      out_shape=jax.ShapeDtypeStruct((1, 10), jnp.float32),
      scratch_shapes=[pltpu.VMEM((1, D), jnp.float32)],
      compiler_params=pltpu.CompilerParams(
          dimension_semantics=("arbitrary",)),
  )(a2, a2, a1, w2, w2, b2.reshape(1, D), wd, bd.reshape(1, HALF),
    wc, bc.reshape(1, 10))


def kernel(in_feat, edge_weights, W1, b1, W2, b2, Wd, bd, Wc, bc, edge_index):
  npad = E_PAD - E
  src = jnp.concatenate([edge_index[0], jnp.zeros((npad,), jnp.int32)])
  dst = jnp.concatenate([edge_index[1], jnp.zeros((npad,), jnp.int32)])
  w = jnp.concatenate([edge_weights, jnp.zeros((npad,), jnp.float32)])
  packed = jnp.stack([src, src + N, dst], axis=0)
  packed = packed.reshape(3, ROWS_PAD, B).transpose(1, 0, 2)
  w_rows = w.reshape(ROWS_PAD, B)

  ones = jnp.ones((N, 1), jnp.float32)
  pad = jnp.zeros((N, AUGW - HALF - 1), jnp.float32)
  table1 = jnp.concatenate([
      jnp.concatenate([in_feat[:, :HALF], ones, pad], axis=1),
      jnp.concatenate([in_feat[:, HALF:], ones, pad], axis=1),
  ], axis=0)

  zeros_aug = jnp.zeros((NPT, AUGW), jnp.float32)
  zeros_half = jnp.zeros((NPT, HALF), jnp.float32)

  a1 = _sc_agg_aug(table1, packed, w_rows, zeros_aug)
  h = _tc_layer1(a1, W1, b1)
  a2 = _sc_agg_half(h, packed, w_rows, zeros_half)
  return _tc_layer2(a2, a1, W2, b2, Wd, bd, Wc, bc)


# speedup vs baseline: 1.5515x; 1.1696x over previous
import functools

import jax
import jax.numpy as jnp
from jax import lax
from jax.experimental import pallas as pl
from jax.experimental.pallas import tpu as pltpu
from jax.experimental.pallas import tpu_sc as plsc

N = 10000
E = 160000
D = 256
HALF = 128
AUGW = 144
NC = 2
NS = 16
LANES = 16
B = 128
NB_PT = 80
E_PAD = NB_PT * NS * B
ROWS_PAD = E_PAD // B
NPT = N // NS
BLK = 1000
NBLK = N // BLK


def _make_sc_aggregate(width):
  mesh = plsc.VectorSubcoreMesh(
      core_axis_name="c", subcore_axis_name="s", num_cores=NC, num_subcores=NS)

  @functools.partial(
      pl.kernel,
      out_type=jax.ShapeDtypeStruct((NC * N, width), jnp.float32),
      mesh=mesh,
      scratch_types=[
          pltpu.VMEM_SHARED((N, width), jnp.float32),
      ] + [pltpu.VMEM((B, width), jnp.float32) for _ in range(2)]
        + [pltpu.VMEM((4, B), jnp.int32) for _ in range(2)]
        + [pltpu.SemaphoreType.DMA for _ in range(4)],
      compiler_params=pltpu.CompilerParams(use_tc_tiling_on_sc=False),
  )
  def agg(table_hbm, packed_hbm, zeros_hbm, out_hbm, acc, *scr):
    rows = scr[0:2]
    idxb = scr[2:4]
    gsem = scr[4:6]
    isem = scr[6:8]
    c = lax.axis_index("c")
    s = lax.axis_index("s")
    base_r = s * NB_PT

    pltpu.sync_copy(zeros_hbm, acc.at[pl.ds(s * NPT, NPT)])
    plsc.subcore_barrier()

    def idx_start(j, b):
      pltpu.async_copy(packed_hbm.at[base_r + b], idxb[j], isem[j])

    def idx_wait(j, b):
      pltpu.make_async_copy(packed_hbm.at[base_r + b], idxb[j],
                            isem[j]).wait()

    def gather_start(j):
      pltpu.async_copy(table_hbm.at[idxb[j].at[c]], rows[j], gsem[j])

    def gather_wait(j):
      pltpu.make_async_copy(table_hbm.at[idxb[j].at[c]], rows[j],
                            gsem[j]).wait()

    def scale(j):
      rj = rows[j]
      wj = idxb[j]

      def mul_chunk(kb, carry):
        kbase = kb * LANES
        wk_vec = wj[3, pl.ds(kbase, LANES)]
        for l in range(LANES):
          wk = lax.bitcast_convert_type(wk_vec[l], jnp.float32)
          for q in range(width // LANES):
            sl = pl.ds(q * LANES, LANES)
            rj[kbase + l, sl] = rj[kbase + l, sl] * wk
        return carry

      lax.fori_loop(0, B // LANES, mul_chunk, 0)

    idx_start(0, 0)
    idx_wait(0, 0)
    gather_start(0)
    idx_start(1, 1)

    def body(i, carry):
      for j in range(2):
        b = 2 * i + j
        p = 1 - j
        gather_wait(j)

        @pl.when(b + 1 < NB_PT)
        def _():
          idx_wait(p, b + 1)
          gather_start(p)

        scale(j)

        pltpu.sync_copy(rows[j], acc.at[idxb[j].at[2]], add=True)

        @pl.when(b + 2 < NB_PT)
        def _():
          idx_start(j, b + 2)
      return carry

    lax.fori_loop(0, NB_PT // 2, body, 0)
    plsc.subcore_barrier()
    pltpu.sync_copy(acc.at[pl.ds(s * NPT, NPT)],
                    out_hbm.at[pl.ds(c * N + s * NPT, NPT)])

  return agg


_sc_agg_aug = _make_sc_aggregate(AUGW)
_sc_agg_half = _make_sc_aggregate(HALF)


def _tc_layer1(a1, w1, b1):
  def body(aa_ref, ab_ref, w1a_ref, w1b_ref, b1_ref, out_ref):
    aa = aa_ref[...]
    ab = ab_ref[...]
    deg = aa[:, HALF:HALF + 1]
    sc = jnp.where(deg > 0.0, 1.0 / deg, 0.0)
    xa = aa[:, :HALF] * sc
    xb = ab[:, :HALF] * sc
    h = (jnp.dot(xa, w1a_ref[...], preferred_element_type=jnp.float32)
         + jnp.dot(xb, w1b_ref[...], preferred_element_type=jnp.float32)
         + b1_ref[...])
    out_ref[...] = jnp.maximum(h, 0.0)

  return pl.pallas_call(
      body,
      grid=(2, NBLK),
      in_specs=[
          pl.BlockSpec((BLK, AUGW), lambda j, i: (i, 0)),
          pl.BlockSpec((BLK, AUGW), lambda j, i: (i + NBLK, 0)),
          pl.BlockSpec((HALF, HALF), lambda j, i: (0, j)),
          pl.BlockSpec((HALF, HALF), lambda j, i: (1, j)),
          pl.BlockSpec((1, HALF), lambda j, i: (0, j)),
      ],
      out_specs=pl.BlockSpec((BLK, HALF), lambda j, i: (j * NBLK + i, 0)),
      out_shape=jax.ShapeDtypeStruct((2 * N, HALF), jnp.float32),
      compiler_params=pltpu.CompilerParams(
          dimension_semantics=("parallel", "parallel")),
  )(a1, a1, w1, w1, b1.reshape(1, D))


def _tc_layer2(a2, a1, w2, b2, wd, bd, wc, bc):
  def body(a2a_ref, a2b_ref, dega_ref, w2a_ref, w2b_ref, b2_ref,
           wd_ref, bd_ref, wc_ref, bc_ref, out_ref, acc_ref):
    i = pl.program_id(0)

    @pl.when(i == 0)
    def _():
      acc_ref[...] = jnp.zeros_like(acc_ref)

    deg = dega_ref[...][:, HALF:HALF + 1]
    sc = jnp.where(deg > 0.0, 1.0 / deg, 0.0)
    xa = a2a_ref[...] * sc
    xb = a2b_ref[...] * sc
    h2 = (jnp.dot(xa, w2a_ref[...], preferred_element_type=jnp.float32)
          + jnp.dot(xb, w2b_ref[...], preferred_element_type=jnp.float32)
          + b2_ref[...])
    h2 = jnp.maximum(h2, 0.0)
    acc_ref[...] += jnp.sum(h2, axis=0, keepdims=True)

    @pl.when(i == NBLK - 1)
    def _():
      hg = acc_ref[...] * (1.0 / N)
      o1 = jnp.maximum(
          jnp.dot(hg, wd_ref[...], preferred_element_type=jnp.float32)
          + bd_ref[...], 0.0)
      out_ref[...] = (
          jnp.dot(o1, wc_ref[...], preferred_element_type=jnp.float32)
          + bc_ref[...])

  return pl.pallas_call(
      body,
      grid=(NBLK,),
      in_specs=[
          pl.BlockSpec((BLK, HALF), lambda i: (i, 0)),
          pl.BlockSpec((BLK, HALF), lambda i: (i + NBLK, 0)),
          pl.BlockSpec((BLK, AUGW), lambda i: (i, 0)),
          pl.BlockSpec((HALF, D), lambda i: (0, 0)),
          pl.BlockSpec((HALF, D), lambda i: (1, 0)),
          pl.BlockSpec((1, D), lambda i: (0, 0)),
          pl.BlockSpec((D, HALF), lambda i: (0, 0)),
          pl.BlockSpec((1, HALF), lambda i: (0, 0)),
          pl.BlockSpec((HALF, 10), lambda i: (0, 0)),
          pl.BlockSpec((1, 10), lambda i: (0, 0)),
      ],
      out_specs=pl.BlockSpec((1, 10), lambda i: (0, 0)),
      out_shape=jax.ShapeDtypeStruct((1, 10), jnp.float32),
      scratch_shapes=[pltpu.VMEM((1, D), jnp.float32)],
      compiler_params=pltpu.CompilerParams(
          dimension_semantics=("arbitrary",)),
  )(a2, a2, a1, w2, w2, b2.reshape(1, D), wd, bd.reshape(1, HALF),
    wc, bc.reshape(1, 10))


def kernel(in_feat, edge_weights, W1, b1, W2, b2, Wd, bd, Wc, bc, edge_index):
  npad = E_PAD - E
  src = jnp.concatenate([edge_index[0], jnp.zeros((npad,), jnp.int32)])
  dst = jnp.concatenate([edge_index[1], jnp.zeros((npad,), jnp.int32)])
  w = jnp.concatenate([edge_weights, jnp.zeros((npad,), jnp.float32)])
  w_bits = lax.bitcast_convert_type(w, jnp.int32)
  packed = jnp.stack([src, src + N, dst, w_bits], axis=0)
  packed = packed.reshape(4, ROWS_PAD, B).transpose(1, 0, 2)

  ones = jnp.ones((N, 1), jnp.float32)
  pad = jnp.zeros((N, AUGW - HALF - 1), jnp.float32)
  table1 = jnp.concatenate([
      jnp.concatenate([in_feat[:, :HALF], ones, pad], axis=1),
      jnp.concatenate([in_feat[:, HALF:], ones, pad], axis=1),
  ], axis=0)

  zeros_aug = jnp.zeros((NPT, AUGW), jnp.float32)
  zeros_half = jnp.zeros((NPT, HALF), jnp.float32)

  a1 = _sc_agg_aug(table1, packed, zeros_aug)
  h = _tc_layer1(a1, W1, b1)
  a2 = _sc_agg_half(h, packed, zeros_half)
  return _tc_layer2(a2, a1, W2, b2, Wd, bd, Wc, bc)
